# baseline (speedup 1.0000x reference)
import functools

import jax
import jax.numpy as jnp
from jax import lax
from jax.experimental import pallas as pl
from jax.experimental.pallas import tpu as pltpu
from jax.experimental.pallas import tpu_sc as plsc

_N = 10000
_K = 32
_HID = 64
_IN = 128

_NC, _NS = 2, 16
_NW = _NC * _NS
_NPW = 320
_NLAST = _N - (_NW - 1) * _NPW
_CPC = 4
_CK = _CPC * _K
_NCH = _NPW // _CPC
_NCH_LAST = _NLAST // _CPC
_NBUF = 4


def _gather_max(table, gidx, D):
    mesh = plsc.VectorSubcoreMesh(
        core_axis_name="c", subcore_axis_name="s",
        num_cores=_NC, num_subcores=_NS)
    lb = 32

    @functools.partial(
        pl.kernel,
        out_type=jax.ShapeDtypeStruct((_N, D), jnp.bfloat16),
        mesh=mesh,
        compiler_params=pltpu.CompilerParams(use_tc_tiling_on_sc=False),
        scratch_types=[
            pltpu.VMEM((_NCH * _CK,), jnp.int32),
            pltpu.VMEM((_NBUF, _CK, D), jnp.bfloat16),
            pltpu.VMEM((_NPW, D), jnp.bfloat16),
            pltpu.VMEM_SHARED((_N, D), jnp.bfloat16),
            pltpu.SemaphoreType.DMA,
            pltpu.SemaphoreType.DMA,
            pltpu.SemaphoreType.DMA,
            pltpu.SemaphoreType.DMA,
        ],
    )
    def gmax(table_hbm, gidx_hbm, out_hbm, gidx_v, rows_v, out_v, table_sh,
             sem0, sem1, sem2, sem3):
        wid = lax.axis_index("s") * _NC + lax.axis_index("c")
        last = wid == _NW - 1
        @pl.when(lax.axis_index("s") == 0)
        def _():
            pltpu.sync_copy(table_hbm, table_sh)

        @pl.when(jnp.logical_not(last))
        def _():
            pltpu.sync_copy(gidx_hbm.at[pl.ds(wid * _NPW * _K, _NPW * _K)],
                            gidx_v.at[pl.ds(0, _NPW * _K)])

        @pl.when(last)
        def _():
            pltpu.sync_copy(gidx_hbm.at[pl.ds(wid * _NPW * _K, _NLAST * _K)],
                            gidx_v.at[pl.ds(0, _NLAST * _K)])

        plsc.subcore_barrier()
        nch = jnp.where(last, _NCH_LAST, _NCH)
        sems = (sem0, sem1, sem2, sem3)

        def dma(g, b):
            return pltpu.make_async_copy(
                table_sh.at[gidx_v.at[pl.ds(g * _CK, _CK)]],
                rows_v.at[b], sems[b])

        for b in range(_NBUF):
            dma(b, b).start()

        def chunk_body(g, b):
            dma(g, b).wait()
            for c in range(_CPC):
                row0 = c * _K
                for dk in range(D // lb):
                    ds = pl.ds(dk * lb, lb)
                    acc = rows_v[b, row0, ds]
                    for k in range(1, _K):
                        acc = jnp.maximum(acc, rows_v[b, row0 + k, ds])
                    out_v[g * _CPC + c, ds] = acc

            @pl.when(g + _NBUF < nch)
            def _():
                dma(g + _NBUF, b).start()

        def body(gq, carry):
            for b in range(_NBUF):
                chunk_body(gq * _NBUF + b, b)
            return carry

        lax.fori_loop(0, nch // _NBUF, body, 0)

        @pl.when(jnp.logical_not(last))
        def _():
            pltpu.sync_copy(out_v.at[pl.ds(0, _NPW)],
                            out_hbm.at[pl.ds(wid * _NPW, _NPW)])

        @pl.when(last)
        def _():
            pltpu.sync_copy(out_v.at[pl.ds(0, _NLAST)],
                            out_hbm.at[pl.ds(wid * _NPW, _NLAST)])

    return gmax(table, gidx)


def _tc_pre(h, x, Wz, Wr, Wq):
    def body(h_ref, x_ref, wz_ref, wr_ref, wq_ref, yzr_ref, yqx_ref):
        h0 = h_ref[0]
        x0 = x_ref[0]
        yz = (jnp.dot(h0, wz_ref[:_HID], preferred_element_type=jnp.float32)
              + jnp.dot(x0, wz_ref[_HID:], preferred_element_type=jnp.float32))
        yr = (jnp.dot(h0, wr_ref[:_HID], preferred_element_type=jnp.float32)
              + jnp.dot(x0, wr_ref[_HID:], preferred_element_type=jnp.float32))
        yzr_ref[...] = jnp.concatenate([yz, yr], axis=1).astype(jnp.bfloat16)
        yqx_ref[...] = jnp.dot(
            x0, wq_ref[_HID:], preferred_element_type=jnp.float32)

    return pl.pallas_call(
        body,
        out_shape=(jax.ShapeDtypeStruct((_N, 2 * _HID), jnp.bfloat16),
                   jax.ShapeDtypeStruct((_N, _HID), jnp.float32)),
    )(h, x, Wz, Wr, Wq)


def _tc_mid(mzr, h, yqx, Wq, bz, br):
    def body(m_ref, h_ref, yqx_ref, wq_ref, bz_ref, br_ref, z_ref, tq_ref):
        m = m_ref[...].astype(jnp.float32)
        z = jax.nn.sigmoid(m[:, :_HID] + bz_ref[...])
        r = jax.nn.sigmoid(m[:, _HID:] + br_ref[...])
        z_ref[...] = z
        tq_ref[...] = (yqx_ref[...] + jnp.dot(
            r * h_ref[0], wq_ref[:_HID], preferred_element_type=jnp.float32)
        ).astype(jnp.bfloat16)

    return pl.pallas_call(
        body,
        out_shape=(jax.ShapeDtypeStruct((_N, _HID), jnp.float32),
                   jax.ShapeDtypeStruct((_N, _HID), jnp.bfloat16)),
    )(mzr, h, yqx, Wq, bz[None, :], br[None, :])


def _tc_post(mq, z, h, bq):
    def body(m_ref, z_ref, h_ref, b_ref, out_ref):
        q = jnp.tanh(m_ref[...].astype(jnp.float32) + b_ref[...])
        z = z_ref[...]
        out_ref[0] = (1.0 - z) * h_ref[0] + z * q

    return pl.pallas_call(
        body,
        out_shape=jax.ShapeDtypeStruct((1, _N, _HID), jnp.float32),
    )(mq, z, h, bq[None, :])


def kernel(h, x, c, graph, Wz, bz, Wr, br, Wq, bq):
    del c
    gidx = graph.reshape(_N * _K)
    yzr, yqx = _tc_pre(h, x, Wz, Wr, Wq)
    mzr = _gather_max(yzr, gidx, 2 * _HID)
    z, tq = _tc_mid(mzr, h, yqx, Wq, bz, br)
    mq = _gather_max(tq, gidx, _HID)
    return _tc_post(mq, z, h, bq)


# speedup vs baseline: 1.1688x; 1.1688x over previous
import functools

import jax
import jax.numpy as jnp
from jax import lax
from jax.experimental import pallas as pl
from jax.experimental.pallas import tpu as pltpu
from jax.experimental.pallas import tpu_sc as plsc

_N = 10000
_K = 32
_HID = 64
_IN = 128

_NC, _NS = 2, 16
_NW = _NC * _NS
_NPW = 320
_NLAST = _N - (_NW - 1) * _NPW
_CPC = 4
_CK = _CPC * _K
_NCH = _NPW // _CPC
_NCH_LAST = _NLAST // _CPC
_NBUF = 2


def _gather_max(table, gidx, D):
    mesh = plsc.VectorSubcoreMesh(
        core_axis_name="c", subcore_axis_name="s",
        num_cores=_NC, num_subcores=_NS)
    lb = 32

    @functools.partial(
        pl.kernel,
        out_type=jax.ShapeDtypeStruct((_N, D), jnp.bfloat16),
        mesh=mesh,
        compiler_params=pltpu.CompilerParams(use_tc_tiling_on_sc=False),
        scratch_types=[
            pltpu.VMEM((_NCH * _CK,), jnp.int32),
            pltpu.VMEM((_NBUF, _CK, D), jnp.bfloat16),
            pltpu.VMEM((_NPW, D), jnp.bfloat16),
            pltpu.VMEM_SHARED((_N, D), jnp.bfloat16),
            pltpu.SemaphoreType.DMA,
            pltpu.SemaphoreType.DMA,
        ],
    )
    def gmax(table_hbm, gidx_hbm, out_hbm, gidx_v, rows_v, out_v, table_sh,
             sem0, sem1):
        wid = lax.axis_index("s") * _NC + lax.axis_index("c")
        last = wid == _NW - 1
        @pl.when(lax.axis_index("s") == 0)
        def _():
            pltpu.sync_copy(table_hbm, table_sh)

        @pl.when(jnp.logical_not(last))
        def _():
            pltpu.sync_copy(gidx_hbm.at[pl.ds(wid * _NPW * _K, _NPW * _K)],
                            gidx_v.at[pl.ds(0, _NPW * _K)])

        @pl.when(last)
        def _():
            pltpu.sync_copy(gidx_hbm.at[pl.ds(wid * _NPW * _K, _NLAST * _K)],
                            gidx_v.at[pl.ds(0, _NLAST * _K)])

        plsc.subcore_barrier()
        nch = jnp.where(last, _NCH_LAST, _NCH)
        sems = (sem0, sem1)

        def dma(g, b):
            return pltpu.make_async_copy(
                table_sh.at[gidx_v.at[pl.ds(g * _CK, _CK)]],
                rows_v.at[b], sems[b])

        for b in range(_NBUF):
            dma(b, b).start()

        def chunk_body(g, b):
            dma(g, b).wait()
            for c in range(_CPC):
                row0 = c * _K
                for dk in range(D // lb):
                    ds = pl.ds(dk * lb, lb)
                    acc = rows_v[b, row0, ds]
                    for k in range(1, _K):
                        acc = jnp.maximum(acc, rows_v[b, row0 + k, ds])
                    out_v[g * _CPC + c, ds] = acc

            @pl.when(g + _NBUF < nch)
            def _():
                dma(g + _NBUF, b).start()

        def body(gq, carry):
            for b in range(_NBUF):
                chunk_body(gq * _NBUF + b, b)
            return carry

        lax.fori_loop(0, nch // _NBUF, body, 0)

        @pl.when(jnp.logical_not(last))
        def _():
            pltpu.sync_copy(out_v.at[pl.ds(0, _NPW)],
                            out_hbm.at[pl.ds(wid * _NPW, _NPW)])

        @pl.when(last)
        def _():
            pltpu.sync_copy(out_v.at[pl.ds(0, _NLAST)],
                            out_hbm.at[pl.ds(wid * _NPW, _NLAST)])

    return gmax(table, gidx)


def _tc_pre(h, x, Wz, Wr, Wq):
    def body(h_ref, x_ref, wz_ref, wr_ref, wq_ref, yzr_ref, yqx_ref):
        h0 = h_ref[0]
        x0 = x_ref[0]
        yz = (jnp.dot(h0, wz_ref[:_HID], preferred_element_type=jnp.float32)
              + jnp.dot(x0, wz_ref[_HID:], preferred_element_type=jnp.float32))
        yr = (jnp.dot(h0, wr_ref[:_HID], preferred_element_type=jnp.float32)
              + jnp.dot(x0, wr_ref[_HID:], preferred_element_type=jnp.float32))
        yzr_ref[...] = jnp.concatenate([yz, yr], axis=1).astype(jnp.bfloat16)
        yqx_ref[...] = jnp.dot(
            x0, wq_ref[_HID:], preferred_element_type=jnp.float32)

    return pl.pallas_call(
        body,
        out_shape=(jax.ShapeDtypeStruct((_N, 2 * _HID), jnp.bfloat16),
                   jax.ShapeDtypeStruct((_N, _HID), jnp.float32)),
    )(h, x, Wz, Wr, Wq)


def _tc_mid(mzr, h, yqx, Wq, bz, br):
    def body(m_ref, h_ref, yqx_ref, wq_ref, bz_ref, br_ref, z_ref, tq_ref):
        m = m_ref[...].astype(jnp.float32)
        z = jax.nn.sigmoid(m[:, :_HID] + bz_ref[...])
        r = jax.nn.sigmoid(m[:, _HID:] + br_ref[...])
        z_ref[...] = z
        tq_ref[...] = (yqx_ref[...] + jnp.dot(
            r * h_ref[0], wq_ref[:_HID], preferred_element_type=jnp.float32)
        ).astype(jnp.bfloat16)

    return pl.pallas_call(
        body,
        out_shape=(jax.ShapeDtypeStruct((_N, _HID), jnp.float32),
                   jax.ShapeDtypeStruct((_N, _HID), jnp.bfloat16)),
    )(mzr, h, yqx, Wq, bz[None, :], br[None, :])


def _tc_post(mq, z, h, bq):
    def body(m_ref, z_ref, h_ref, b_ref, out_ref):
        q = jnp.tanh(m_ref[...].astype(jnp.float32) + b_ref[...])
        z = z_ref[...]
        out_ref[0] = (1.0 - z) * h_ref[0] + z * q

    return pl.pallas_call(
        body,
        out_shape=jax.ShapeDtypeStruct((1, _N, _HID), jnp.float32),
    )(mq, z, h, bq[None, :])


def kernel(h, x, c, graph, Wz, bz, Wr, br, Wq, bq):
    del c
    gidx = graph.reshape(_N * _K)
    yzr, yqx = _tc_pre(h, x, Wz, Wr, Wq)
    mzr = _gather_max(yzr, gidx, 2 * _HID)
    z, tq = _tc_mid(mzr, h, yqx, Wq, bz, br)
    mq = _gather_max(tq, gidx, _HID)
    return _tc_post(mq, z, h, bq)
